# Initial kernel scaffold; baseline (speedup 1.0000x reference)
import functools

import jax
import jax.numpy as jnp
from jax import lax
from jax.experimental import pallas as pl
from jax.experimental.pallas import tpu as pltpu
from jax.experimental.pallas import tpu_sc as plsc

N = 10000
E = 320000
D = 128
DE = 16
ALPHA = 0.5

NC = 2
NS = 16
NW = NC * NS
EPW = E // NW
C = 80
NCHUNK = EPW // C
RPS = N // NS


def _tables_body(x_ref, ws_ref, wd_ref, ps_ref, pd_ref):
    x = x_ref[...]
    ps_ref[...] = jnp.dot(x, ws_ref[...], preferred_element_type=jnp.float32)
    pd_ref[...] = jnp.dot(x, wd_ref[...], preferred_element_type=jnp.float32)


def _edge_tables(x, ws, wd):
    return pl.pallas_call(
        _tables_body,
        out_shape=(
            jax.ShapeDtypeStruct((N, DE), jnp.float32),
            jax.ShapeDtypeStruct((N, DE), jnp.float32),
        ),
    )(x, ws, wd)


_QBLK = 20000


def _q_body(ea_ref, we_ref, be_ref, q_ref):
    q_ref[...] = (
        jnp.dot(ea_ref[...], we_ref[...], preferred_element_type=jnp.float32)
        + be_ref[...]
    )


def _edge_q(ea, we, be):
    grid = E // _QBLK
    return pl.pallas_call(
        _q_body,
        grid=(grid,),
        in_specs=[
            pl.BlockSpec((_QBLK, DE), lambda i: (i, 0)),
            pl.BlockSpec((DE, DE), lambda i: (0, 0)),
            pl.BlockSpec((1, DE), lambda i: (0, 0)),
        ],
        out_specs=pl.BlockSpec((_QBLK, DE), lambda i: (i, 0)),
        out_shape=jax.ShapeDtypeStruct((E, DE), jnp.float32),
    )(ea, we, be.reshape(1, DE))


def _node_body(x_ref, part_ref, wx_ref, wa_ref, bn_ref, xo_ref):
    x = x_ref[...]
    agg = part_ref[0] + part_ref[1]
    dx = jnp.dot(x, wx_ref[...], preferred_element_type=jnp.float32)
    dx = dx + jnp.dot(agg, wa_ref[...], preferred_element_type=jnp.float32)
    dx = jnp.maximum(dx + bn_ref[...], 0.0)
    sa = jnp.float32(ALPHA) ** 0.5
    sb = jnp.float32(1.0 - ALPHA) ** 0.5
    xo_ref[...] = sa * dx + sb * x


def _node_update(x, partials, wx, wa, bn):
    return pl.pallas_call(
        _node_body,
        out_shape=jax.ShapeDtypeStruct((N, D), jnp.float32),
    )(x, partials, wx, wa, bn.reshape(1, D))


def _sc_edge_body(ps_hbm, pd_hbm, q_hbm, src_hbm, dst_hbm, zeros_hbm,
                  ea_hbm, part_hbm,
                  idx_s, idx_d, rows_s, rows_d, q_v, out_v, agg_sp,
                  sem_s, sem_d):
    cid = lax.axis_index("c")
    sid = lax.axis_index("s")
    wid = sid * NC + cid

    pltpu.sync_copy(zeros_hbm.at[pl.ds(sid * RPS, RPS)],
                    agg_sp.at[pl.ds(sid * RPS, RPS)])
    plsc.subcore_barrier()

    base0 = wid * EPW

    def chunk(c, carry):
        base = base0 + c * C
        pltpu.sync_copy(src_hbm.at[pl.ds(base, C)], idx_s)
        pltpu.sync_copy(dst_hbm.at[pl.ds(base, C)], idx_d)
        cp_s = pltpu.async_copy(ps_hbm.at[idx_s], rows_s, sem_s)
        cp_d = pltpu.async_copy(pd_hbm.at[idx_d], rows_d, sem_d)
        pltpu.sync_copy(q_hbm.at[pl.ds(base, C)], q_v)
        cp_s.wait()
        cp_d.wait()

        def edge(e, carry2):
            v = rows_s[e] + rows_d[e] + q_v[e]
            out_v[e] = jnp.maximum(v, 0.0)
            return carry2

        lax.fori_loop(0, C, edge, 0, unroll=4)
        pltpu.sync_copy(out_v, ea_hbm.at[pl.ds(base, C)])
        pltpu.sync_copy(out_v, agg_sp.at[idx_d], add=True)
        return carry

    lax.fori_loop(0, NCHUNK, chunk, 0)

    plsc.subcore_barrier()
    pltpu.sync_copy(agg_sp.at[pl.ds(sid * RPS, RPS)],
                    part_hbm.at[cid].at[pl.ds(sid * RPS, RPS)])


@functools.lru_cache(maxsize=None)
def _sc_edge_call():
    mesh = plsc.VectorSubcoreMesh(core_axis_name="c", subcore_axis_name="s")
    return pl.kernel(
        _sc_edge_body,
        out_type=(
            jax.ShapeDtypeStruct((E, DE), jnp.float32),
            jax.ShapeDtypeStruct((NC, N, DE), jnp.float32),
        ),
        mesh=mesh,
        scratch_types=[
            pltpu.VMEM((C,), jnp.int32),
            pltpu.VMEM((C,), jnp.int32),
            pltpu.VMEM((C, DE), jnp.float32),
            pltpu.VMEM((C, DE), jnp.float32),
            pltpu.VMEM((C, DE), jnp.float32),
            pltpu.VMEM((C, DE), jnp.float32),
            pltpu.VMEM_SHARED((N, DE), jnp.float32),
            pltpu.SemaphoreType.DMA,
            pltpu.SemaphoreType.DMA,
        ],
    )


def kernel(x, edge_index, edge_attr, We1, be1, Wn1, bn1, We2, be2, Wn2, bn2):
    src = edge_index[0]
    dst = edge_index[1]
    zeros = jnp.zeros((N, DE), jnp.float32)
    sc_call = _sc_edge_call()

    ea = edge_attr
    edge_attrs = [edge_attr]
    for We, be, Wn, bn in ((We1, be1, Wn1, bn1), (We2, be2, Wn2, bn2)):
        ps, pd = _edge_tables(x, We[:D], We[D:2 * D])
        q = _edge_q(ea, We[2 * D:], be)
        ea, partials = sc_call(ps, pd, q, src, dst, zeros)
        x = _node_update(x, partials, Wn[:D], Wn[D:], bn)
        edge_attrs.append(ea)

    return x, ea, jnp.concatenate(edge_attrs, axis=1)


# speedup vs baseline: 3.0308x; 3.0308x over previous
import functools

import jax
import jax.numpy as jnp
from jax import lax
from jax.experimental import pallas as pl
from jax.experimental.pallas import tpu as pltpu
from jax.experimental.pallas import tpu_sc as plsc

N = 10000
E = 320000
D = 128
DE = 16
ALPHA = 0.5

NC = 2
NS = 16
NW = NC * NS
EPW = E // NW
C = 80
NCHUNK = EPW // C
NINIT = 10
RPS = N // NINIT


def _tables_body(x_ref, ws_ref, wd_ref, ps_ref, pd_ref):
    x = x_ref[...]
    ps_ref[...] = jnp.dot(x, ws_ref[...], preferred_element_type=jnp.float32)
    pd_ref[...] = jnp.dot(x, wd_ref[...], preferred_element_type=jnp.float32)


def _edge_tables(x, ws, wd):
    return pl.pallas_call(
        _tables_body,
        out_shape=(
            jax.ShapeDtypeStruct((N, DE), jnp.float32),
            jax.ShapeDtypeStruct((N, DE), jnp.float32),
        ),
    )(x, ws, wd)


_QBLK = 20000


def _q_body(ea_ref, we_ref, be_ref, q_ref):
    q_ref[...] = (
        jnp.dot(ea_ref[...], we_ref[...], preferred_element_type=jnp.float32)
        + be_ref[...]
    )


def _edge_q(ea, we, be):
    grid = E // _QBLK
    return pl.pallas_call(
        _q_body,
        grid=(grid,),
        in_specs=[
            pl.BlockSpec((_QBLK, DE), lambda i: (i, 0)),
            pl.BlockSpec((DE, DE), lambda i: (0, 0)),
            pl.BlockSpec((1, DE), lambda i: (0, 0)),
        ],
        out_specs=pl.BlockSpec((_QBLK, DE), lambda i: (i, 0)),
        out_shape=jax.ShapeDtypeStruct((E, DE), jnp.float32),
    )(ea, we, be.reshape(1, DE))


def _node_body(x_ref, part_ref, wx_ref, wa_ref, bn_ref, xo_ref):
    x = x_ref[...]
    agg = part_ref[0] + part_ref[1]
    dx = jnp.dot(x, wx_ref[...], preferred_element_type=jnp.float32)
    dx = dx + jnp.dot(agg, wa_ref[...], preferred_element_type=jnp.float32)
    dx = jnp.maximum(dx + bn_ref[...], 0.0)
    sa = jnp.float32(ALPHA) ** 0.5
    sb = jnp.float32(1.0 - ALPHA) ** 0.5
    xo_ref[...] = sa * dx + sb * x


def _node_update(x, partials, wx, wa, bn):
    return pl.pallas_call(
        _node_body,
        out_shape=jax.ShapeDtypeStruct((N, D), jnp.float32),
    )(x, partials, wx, wa, bn.reshape(1, D))


def _sc_edge_body(ps_hbm, pd_hbm, q_hbm, src_hbm, dst_hbm, zeros_hbm,
                  ea_hbm, part_hbm,
                  idx_s, idx_d, rows_s, rows_d, q_v, out_v, agg_sp,
                  sem_s, sem_d):
    cid = lax.axis_index("c")
    sid = lax.axis_index("s")
    wid = sid * NC + cid

    @pl.when(sid < NINIT)
    def _init():
        pltpu.sync_copy(zeros_hbm.at[pl.ds(sid * RPS, RPS)],
                        agg_sp.at[pl.ds(sid * RPS, RPS)])

    plsc.subcore_barrier()

    base0 = wid * EPW

    def chunk(c, carry):
        base = base0 + c * C
        pltpu.sync_copy(src_hbm.at[pl.ds(base, C)], idx_s)
        pltpu.sync_copy(dst_hbm.at[pl.ds(base, C)], idx_d)
        cp_s = pltpu.async_copy(ps_hbm.at[idx_s], rows_s, sem_s)
        cp_d = pltpu.async_copy(pd_hbm.at[idx_d], rows_d, sem_d)
        pltpu.sync_copy(q_hbm.at[pl.ds(base, C)], q_v)
        cp_s.wait()
        cp_d.wait()

        def edge(e, carry2):
            v = rows_s[e] + rows_d[e] + q_v[e]
            out_v[e] = jnp.maximum(v, 0.0)
            return carry2

        lax.fori_loop(0, C, edge, 0, unroll=4)
        pltpu.sync_copy(out_v, ea_hbm.at[pl.ds(base, C)])
        pltpu.sync_copy(out_v, agg_sp.at[idx_d], add=True)
        return carry

    lax.fori_loop(0, NCHUNK, chunk, 0)

    plsc.subcore_barrier()

    @pl.when(sid < NINIT)
    def _drain():
        pltpu.sync_copy(agg_sp.at[pl.ds(sid * RPS, RPS)],
                        part_hbm.at[cid].at[pl.ds(sid * RPS, RPS)])


@functools.lru_cache(maxsize=None)
def _sc_edge_call():
    mesh = plsc.VectorSubcoreMesh(core_axis_name="c", subcore_axis_name="s")
    return pl.kernel(
        _sc_edge_body,
        out_type=(
            jax.ShapeDtypeStruct((E, DE), jnp.float32),
            jax.ShapeDtypeStruct((NC, N, DE), jnp.float32),
        ),
        mesh=mesh,
        compiler_params=pltpu.CompilerParams(use_tc_tiling_on_sc=False),
        scratch_types=[
            pltpu.VMEM((C,), jnp.int32),
            pltpu.VMEM((C,), jnp.int32),
            pltpu.VMEM((C, DE), jnp.float32),
            pltpu.VMEM((C, DE), jnp.float32),
            pltpu.VMEM((C, DE), jnp.float32),
            pltpu.VMEM((C, DE), jnp.float32),
            pltpu.VMEM_SHARED((N, DE), jnp.float32),
            pltpu.SemaphoreType.DMA,
            pltpu.SemaphoreType.DMA,
        ],
    )


def kernel(x, edge_index, edge_attr, We1, be1, Wn1, bn1, We2, be2, Wn2, bn2):
    src = edge_index[0]
    dst = edge_index[1]
    zeros = jnp.zeros((N, DE), jnp.float32)
    sc_call = _sc_edge_call()

    ea = edge_attr
    edge_attrs = [edge_attr]
    for We, be, Wn, bn in ((We1, be1, Wn1, bn1), (We2, be2, Wn2, bn2)):
        ps, pd = _edge_tables(x, We[:D], We[D:2 * D])
        q = _edge_q(ea, We[2 * D:], be)
        ea, partials = sc_call(ps, pd, q, src, dst, zeros)
        x = _node_update(x, partials, Wn[:D], Wn[D:], bn)
        edge_attrs.append(ea)

    return x, ea, jnp.concatenate(edge_attrs, axis=1)


# speedup vs baseline: 3.5788x; 1.1808x over previous
import functools

import jax
import jax.numpy as jnp
from jax import lax
from jax.experimental import pallas as pl
from jax.experimental.pallas import tpu as pltpu
from jax.experimental.pallas import tpu_sc as plsc

N = 10000
E = 320000
D = 128
DE = 16
ALPHA = 0.5

NC = 2
NS = 16
NW = NC * NS
EPW = E // NW
C = 80
NCHUNK = EPW // C
NINIT = 10
RPS = N // NINIT


def _tables_body(x_ref, ws_ref, wd_ref, ps_ref, pd_ref):
    x = x_ref[...]
    ps_ref[...] = jnp.dot(x, ws_ref[...], preferred_element_type=jnp.float32)
    pd_ref[...] = jnp.dot(x, wd_ref[...], preferred_element_type=jnp.float32)


def _edge_tables(x, ws, wd):
    return pl.pallas_call(
        _tables_body,
        out_shape=(
            jax.ShapeDtypeStruct((N, DE), jnp.float32),
            jax.ShapeDtypeStruct((N, DE), jnp.float32),
        ),
    )(x, ws, wd)


E8 = E // 8
_QBLK = 5000


def _q_body(ea_ref, we_ref, be_ref, q_ref):
    q_ref[...] = (
        jnp.dot(ea_ref[...], we_ref[...], preferred_element_type=jnp.float32)
        + be_ref[...]
    )


def _edge_q(ea_p, we_bd, be_t):
    grid = E8 // _QBLK
    return pl.pallas_call(
        _q_body,
        grid=(grid,),
        in_specs=[
            pl.BlockSpec((_QBLK, D), lambda i: (i, 0)),
            pl.BlockSpec((D, D), lambda i: (0, 0)),
            pl.BlockSpec((1, D), lambda i: (0, 0)),
        ],
        out_specs=pl.BlockSpec((_QBLK, D), lambda i: (i, 0)),
        out_shape=jax.ShapeDtypeStruct((E8, D), jnp.float32),
    )(ea_p, we_bd, be_t.reshape(1, D))


_CATBLK = 4000


def _cat_body(a_ref, b_ref, c_ref, o_ref):
    for j in range(8):
        o_ref[:, 48 * j:48 * j + 16] = a_ref[:, 16 * j:16 * j + 16]
        o_ref[:, 48 * j + 16:48 * j + 32] = b_ref[:, 16 * j:16 * j + 16]
        o_ref[:, 48 * j + 32:48 * j + 48] = c_ref[:, 16 * j:16 * j + 16]


def _edge_cat(a_p, b_p, c_p):
    grid = E8 // _CATBLK
    return pl.pallas_call(
        _cat_body,
        grid=(grid,),
        in_specs=[
            pl.BlockSpec((_CATBLK, D), lambda i: (i, 0)),
            pl.BlockSpec((_CATBLK, D), lambda i: (i, 0)),
            pl.BlockSpec((_CATBLK, D), lambda i: (i, 0)),
        ],
        out_specs=pl.BlockSpec((_CATBLK, 3 * D), lambda i: (i, 0)),
        out_shape=jax.ShapeDtypeStruct((E8, 3 * D), jnp.float32),
    )(a_p, b_p, c_p)


def _node_body(x_ref, part_ref, wx_ref, wa_ref, bn_ref, xo_ref):
    x = x_ref[...]
    agg = part_ref[0] + part_ref[1]
    dx = jnp.dot(x, wx_ref[...], preferred_element_type=jnp.float32)
    dx = dx + jnp.dot(agg, wa_ref[...], preferred_element_type=jnp.float32)
    dx = jnp.maximum(dx + bn_ref[...], 0.0)
    sa = jnp.float32(ALPHA) ** 0.5
    sb = jnp.float32(1.0 - ALPHA) ** 0.5
    xo_ref[...] = sa * dx + sb * x


def _node_update(x, partials, wx, wa, bn):
    return pl.pallas_call(
        _node_body,
        out_shape=jax.ShapeDtypeStruct((N, D), jnp.float32),
    )(x, partials, wx, wa, bn.reshape(1, D))


def _sc_edge_body(ps_hbm, pd_hbm, q_hbm, src_hbm, dst_hbm, zeros_hbm,
                  ea_hbm, part_hbm,
                  idx_s, idx_d, rows_s, rows_d, q_v, out_v, agg_sp,
                  sem_s, sem_d):
    cid = lax.axis_index("c")
    sid = lax.axis_index("s")
    wid = sid * NC + cid

    @pl.when(sid < NINIT)
    def _init():
        pltpu.sync_copy(zeros_hbm.at[pl.ds(sid * RPS, RPS)],
                        agg_sp.at[pl.ds(sid * RPS, RPS)])

    plsc.subcore_barrier()

    base0 = wid * EPW

    def chunk(c, carry):
        base = base0 + c * C
        pltpu.sync_copy(src_hbm.at[pl.ds(base, C)], idx_s)
        pltpu.sync_copy(dst_hbm.at[pl.ds(base, C)], idx_d)
        cp_s = pltpu.async_copy(ps_hbm.at[idx_s], rows_s, sem_s)
        cp_d = pltpu.async_copy(pd_hbm.at[idx_d], rows_d, sem_d)
        pltpu.sync_copy(q_hbm.at[pl.ds(base, C)], q_v)
        cp_s.wait()
        cp_d.wait()

        def edge(e, carry2):
            v = rows_s[e] + rows_d[e] + q_v[e]
            out_v[e] = jnp.maximum(v, 0.0)
            return carry2

        lax.fori_loop(0, C, edge, 0, unroll=4)
        pltpu.sync_copy(out_v, ea_hbm.at[pl.ds(base, C)])
        pltpu.sync_copy(out_v, agg_sp.at[idx_d], add=True)
        return carry

    lax.fori_loop(0, NCHUNK, chunk, 0)

    plsc.subcore_barrier()

    @pl.when(sid < NINIT)
    def _drain():
        pltpu.sync_copy(agg_sp.at[pl.ds(sid * RPS, RPS)],
                        part_hbm.at[cid].at[pl.ds(sid * RPS, RPS)])


@functools.lru_cache(maxsize=None)
def _sc_edge_call():
    mesh = plsc.VectorSubcoreMesh(core_axis_name="c", subcore_axis_name="s")
    return pl.kernel(
        _sc_edge_body,
        out_type=(
            jax.ShapeDtypeStruct((E, DE), jnp.float32),
            jax.ShapeDtypeStruct((NC, N, DE), jnp.float32),
        ),
        mesh=mesh,
        compiler_params=pltpu.CompilerParams(use_tc_tiling_on_sc=False),
        scratch_types=[
            pltpu.VMEM((C,), jnp.int32),
            pltpu.VMEM((C,), jnp.int32),
            pltpu.VMEM((C, DE), jnp.float32),
            pltpu.VMEM((C, DE), jnp.float32),
            pltpu.VMEM((C, DE), jnp.float32),
            pltpu.VMEM((C, DE), jnp.float32),
            pltpu.VMEM_SHARED((N, DE), jnp.float32),
            pltpu.SemaphoreType.DMA,
            pltpu.SemaphoreType.DMA,
        ],
    )


def kernel(x, edge_index, edge_attr, We1, be1, Wn1, bn1, We2, be2, Wn2, bn2):
    src = edge_index[0]
    dst = edge_index[1]
    zeros = jnp.zeros((N, DE), jnp.float32)
    sc_call = _sc_edge_call()
    eye8 = jnp.eye(8, dtype=jnp.float32)

    ea_p = edge_attr.reshape(E8, D)
    packed = [ea_p]
    for We, be, Wn, bn in ((We1, be1, Wn1, bn1), (We2, be2, Wn2, bn2)):
        ps, pd = _edge_tables(x, We[:D], We[D:2 * D])
        q_p = _edge_q(ea_p, jnp.kron(eye8, We[2 * D:]), jnp.tile(be, 8))
        ea, partials = sc_call(ps, pd, q_p.reshape(E, DE), src, dst, zeros)
        x = _node_update(x, partials, Wn[:D], Wn[D:], bn)
        ea_p = ea.reshape(E8, D)
        packed.append(ea_p)

    cat = _edge_cat(*packed).reshape(E, 3 * DE)
    return x, ea, cat
